# initial kernel scaffold (unmeasured)
import jax
import jax.numpy as jnp
from jax import lax
from jax.experimental import pallas as pl
from jax.experimental.pallas import tpu as pltpu

N_DEV = 8
B = 2
SQ = 256
DMODEL = 512
DOUT = 512
HQ = 4
DH = 64
HD = HQ * DH
SKV_SH = 256
WIN = 128
KV_KEEP = SQ + WIN
REL1 = KV_KEEP - SKV_SH
SCALE = 0.125
NEG = -1e9


def kernel(x, Wq, K_ext, V_ext, Wo):
    K2 = K_ext.reshape(B, SKV_SH, HD)
    V2 = V_ext.reshape(B, SKV_SH, HD)

    def body(x_ref, wq_ref, k_ref, v_ref, wo_ref, out_ref,
             kall, vall, p1_send, p1_recv, bc_send, bc_recv):
        pos = lax.axis_index("i")

        @pl.when(pos == 0)
        def _():
            sk = pltpu.make_async_remote_copy(
                src_ref=k_ref,
                dst_ref=kall.at[:, pl.ds(0, SKV_SH), :],
                send_sem=p1_send.at[0], recv_sem=p1_recv.at[0],
                device_id=(1,), device_id_type=pl.DeviceIdType.MESH)
            sv = pltpu.make_async_remote_copy(
                src_ref=v_ref,
                dst_ref=vall.at[:, pl.ds(0, SKV_SH), :],
                send_sem=p1_send.at[1], recv_sem=p1_recv.at[1],
                device_id=(1,), device_id_type=pl.DeviceIdType.MESH)
            sk.start()
            sv.start()
            kall[:, 0:SKV_SH, :] = k_ref[...]
            vall[:, 0:SKV_SH, :] = v_ref[...]
            rk = pltpu.make_async_remote_copy(
                src_ref=k_ref.at[:, pl.ds(0, REL1), :],
                dst_ref=kall.at[:, pl.ds(SKV_SH, REL1), :],
                send_sem=p1_send.at[0], recv_sem=p1_recv.at[0],
                device_id=(1,), device_id_type=pl.DeviceIdType.MESH)
            rv = pltpu.make_async_remote_copy(
                src_ref=v_ref.at[:, pl.ds(0, REL1), :],
                dst_ref=vall.at[:, pl.ds(SKV_SH, REL1), :],
                send_sem=p1_send.at[1], recv_sem=p1_recv.at[1],
                device_id=(1,), device_id_type=pl.DeviceIdType.MESH)
            rk.wait_recv()
            rv.wait_recv()
            sk.wait_send()
            sv.wait_send()

        @pl.when(pos == 1)
        def _():
            sk = pltpu.make_async_remote_copy(
                src_ref=k_ref.at[:, pl.ds(0, REL1), :],
                dst_ref=kall.at[:, pl.ds(SKV_SH, REL1), :],
                send_sem=p1_send.at[0], recv_sem=p1_recv.at[0],
                device_id=(0,), device_id_type=pl.DeviceIdType.MESH)
            sv = pltpu.make_async_remote_copy(
                src_ref=v_ref.at[:, pl.ds(0, REL1), :],
                dst_ref=vall.at[:, pl.ds(SKV_SH, REL1), :],
                send_sem=p1_send.at[1], recv_sem=p1_recv.at[1],
                device_id=(0,), device_id_type=pl.DeviceIdType.MESH)
            sk.start()
            sv.start()
            kall[:, SKV_SH:KV_KEEP, :] = k_ref[:, 0:REL1, :]
            vall[:, SKV_SH:KV_KEEP, :] = v_ref[:, 0:REL1, :]
            rk = pltpu.make_async_remote_copy(
                src_ref=k_ref,
                dst_ref=kall.at[:, pl.ds(0, SKV_SH), :],
                send_sem=p1_send.at[0], recv_sem=p1_recv.at[0],
                device_id=(0,), device_id_type=pl.DeviceIdType.MESH)
            rv = pltpu.make_async_remote_copy(
                src_ref=v_ref,
                dst_ref=vall.at[:, pl.ds(0, SKV_SH), :],
                send_sem=p1_send.at[1], recv_sem=p1_recv.at[1],
                device_id=(0,), device_id_type=pl.DeviceIdType.MESH)
            rk.wait_recv()
            rv.wait_recv()
            sk.wait_send()
            sv.wait_send()

        @pl.when(pos <= 1)
        def _():
            qi = lax.broadcasted_iota(jnp.int32, (SQ, KV_KEEP), 0)
            ki = lax.broadcasted_iota(jnp.int32, (SQ, KV_KEEP), 1)
            mask = jnp.abs(qi - ki) <= WIN
            for b in range(B):
                q2d = jnp.dot(x_ref[b], wq_ref[...],
                              preferred_element_type=jnp.float32)
                ctx_heads = []
                for h in range(HQ):
                    qh = q2d[:, h * DH:(h + 1) * DH]
                    kh = kall[b, :, h * DH:(h + 1) * DH]
                    vh = vall[b, :, h * DH:(h + 1) * DH]
                    s = lax.dot_general(
                        qh, kh, (((1,), (1,)), ((), ())),
                        preferred_element_type=jnp.float32) * SCALE
                    s = jnp.where(mask, s, NEG)
                    m = jnp.max(s, axis=-1, keepdims=True)
                    w = jnp.exp(s - m)
                    l = jnp.sum(w, axis=-1, keepdims=True)
                    ctx_heads.append(
                        jnp.dot(w, vh, preferred_element_type=jnp.float32) / l)
                ctx = jnp.concatenate(ctx_heads, axis=-1)
                out_ref[b] = jnp.dot(ctx, wo_ref[...],
                                     preferred_element_type=jnp.float32)

        has_recv = pos >= 2
        has_send = jnp.logical_or(pos <= 3, pos >= 6)
        recv_from = jnp.where(pos <= 4, pos - 1, (pos + 1) % N_DEV)
        send_to = jnp.where(pos == 0, N_DEV - 1,
                            jnp.where(pos <= 3, pos + 1, pos - 1))

        @pl.when(has_recv)
        def _():
            r = pltpu.make_async_remote_copy(
                src_ref=out_ref, dst_ref=out_ref,
                send_sem=bc_send, recv_sem=bc_recv,
                device_id=(recv_from,), device_id_type=pl.DeviceIdType.MESH)
            r.wait_recv()

        @pl.when(has_send)
        def _():
            s = pltpu.make_async_remote_copy(
                src_ref=out_ref, dst_ref=out_ref,
                send_sem=bc_send, recv_sem=bc_recv,
                device_id=(send_to,), device_id_type=pl.DeviceIdType.MESH)
            s.start()
            s.wait_send()

    return pl.pallas_call(
        body,
        out_shape=jax.ShapeDtypeStruct((B, SQ, DOUT), jnp.float32),
        in_specs=[pl.BlockSpec(memory_space=pltpu.VMEM)] * 5,
        out_specs=pl.BlockSpec(memory_space=pltpu.VMEM),
        scratch_shapes=[
            pltpu.VMEM((B, KV_KEEP, HD), jnp.float32),
            pltpu.VMEM((B, KV_KEEP, HD), jnp.float32),
            pltpu.SemaphoreType.DMA((2,)),
            pltpu.SemaphoreType.DMA((2,)),
            pltpu.SemaphoreType.DMA,
            pltpu.SemaphoreType.DMA,
        ],
        compiler_params=pltpu.CompilerParams(collective_id=0),
    )(x, Wq, K2, V2, Wo)


# baseline (device time: 62196 ns/iter reference)
import jax
import jax.numpy as jnp
from jax import lax
from jax.experimental import pallas as pl
from jax.experimental.pallas import tpu as pltpu

N_DEV = 8
B = 2
SQ = 256
DMODEL = 512
DOUT = 512
HQ = 4
DH = 64
HD = HQ * DH
SKV_SH = 256
WIN = 128
KV_KEEP = SQ + WIN
REL1 = KV_KEEP - SKV_SH
SCALE = 0.125
NEG = -1e9


def kernel(x, Wq, K_ext, V_ext, Wo):
    K2 = K_ext.reshape(B, SKV_SH, HD)
    V2 = V_ext.reshape(B, SKV_SH, HD)

    def body(x_ref, wq_ref, k_ref, v_ref, wo_ref, out_ref,
             kall, vall, p1_send, p1_recv, bc_send, bc_recv):
        pos = lax.axis_index("i")

        @pl.when(pos == 0)
        def _():
            sk = pltpu.make_async_remote_copy(
                src_ref=k_ref,
                dst_ref=kall.at[:, pl.ds(0, SKV_SH), :],
                send_sem=p1_send.at[0], recv_sem=p1_recv.at[0],
                device_id=(1,), device_id_type=pl.DeviceIdType.MESH)
            sv = pltpu.make_async_remote_copy(
                src_ref=v_ref,
                dst_ref=vall.at[:, pl.ds(0, SKV_SH), :],
                send_sem=p1_send.at[1], recv_sem=p1_recv.at[1],
                device_id=(1,), device_id_type=pl.DeviceIdType.MESH)
            sk.start()
            sv.start()
            kall[:, 0:SKV_SH, :] = k_ref[...]
            vall[:, 0:SKV_SH, :] = v_ref[...]
            rk = pltpu.make_async_remote_copy(
                src_ref=k_ref.at[:, pl.ds(0, REL1), :],
                dst_ref=kall.at[:, pl.ds(SKV_SH, REL1), :],
                send_sem=p1_send.at[0], recv_sem=p1_recv.at[0],
                device_id=(1,), device_id_type=pl.DeviceIdType.MESH)
            rv = pltpu.make_async_remote_copy(
                src_ref=v_ref.at[:, pl.ds(0, REL1), :],
                dst_ref=vall.at[:, pl.ds(SKV_SH, REL1), :],
                send_sem=p1_send.at[1], recv_sem=p1_recv.at[1],
                device_id=(1,), device_id_type=pl.DeviceIdType.MESH)
            rk.wait_recv()
            rv.wait_recv()
            sk.wait_send()
            sv.wait_send()

        @pl.when(pos == 1)
        def _():
            sk = pltpu.make_async_remote_copy(
                src_ref=k_ref.at[:, pl.ds(0, REL1), :],
                dst_ref=kall.at[:, pl.ds(SKV_SH, REL1), :],
                send_sem=p1_send.at[0], recv_sem=p1_recv.at[0],
                device_id=(0,), device_id_type=pl.DeviceIdType.MESH)
            sv = pltpu.make_async_remote_copy(
                src_ref=v_ref.at[:, pl.ds(0, REL1), :],
                dst_ref=vall.at[:, pl.ds(SKV_SH, REL1), :],
                send_sem=p1_send.at[1], recv_sem=p1_recv.at[1],
                device_id=(0,), device_id_type=pl.DeviceIdType.MESH)
            sk.start()
            sv.start()
            kall[:, SKV_SH:KV_KEEP, :] = k_ref[:, 0:REL1, :]
            vall[:, SKV_SH:KV_KEEP, :] = v_ref[:, 0:REL1, :]
            rk = pltpu.make_async_remote_copy(
                src_ref=k_ref,
                dst_ref=kall.at[:, pl.ds(0, SKV_SH), :],
                send_sem=p1_send.at[0], recv_sem=p1_recv.at[0],
                device_id=(0,), device_id_type=pl.DeviceIdType.MESH)
            rv = pltpu.make_async_remote_copy(
                src_ref=v_ref,
                dst_ref=vall.at[:, pl.ds(0, SKV_SH), :],
                send_sem=p1_send.at[1], recv_sem=p1_recv.at[1],
                device_id=(0,), device_id_type=pl.DeviceIdType.MESH)
            rk.wait_recv()
            rv.wait_recv()
            sk.wait_send()
            sv.wait_send()

        @pl.when(pos <= 1)
        def _():
            qi = lax.broadcasted_iota(jnp.int32, (SQ, KV_KEEP), 0)
            ki = lax.broadcasted_iota(jnp.int32, (SQ, KV_KEEP), 1)
            mask = jnp.abs(qi - ki) <= WIN
            for b in range(B):
                q2d = jnp.dot(x_ref[b], wq_ref[...],
                              preferred_element_type=jnp.float32)
                ctx_heads = []
                for h in range(HQ):
                    qh = q2d[:, h * DH:(h + 1) * DH]
                    kh = kall[b, :, h * DH:(h + 1) * DH]
                    vh = vall[b, :, h * DH:(h + 1) * DH]
                    s = lax.dot_general(
                        qh, kh, (((1,), (1,)), ((), ())),
                        preferred_element_type=jnp.float32) * SCALE
                    s = jnp.where(mask, s, NEG)
                    m = jnp.max(s, axis=-1, keepdims=True)
                    w = jnp.exp(s - m)
                    l = jnp.sum(w, axis=-1, keepdims=True)
                    ctx_heads.append(
                        jnp.dot(w, vh, preferred_element_type=jnp.float32) / l)
                ctx = jnp.concatenate(ctx_heads, axis=-1)
                out_ref[b] = jnp.dot(ctx, wo_ref[...],
                                     preferred_element_type=jnp.float32)

        has_recv = pos >= 2
        has_send = jnp.logical_or(pos <= 3, pos >= 6)
        recv_from = jnp.where(pos <= 4, pos - 1, (pos + 1) % N_DEV)
        send_to = jnp.where(pos == 0, N_DEV - 1,
                            jnp.where(pos <= 3, pos + 1, pos - 1))

        @pl.when(has_recv)
        def _():
            r = pltpu.make_async_remote_copy(
                src_ref=out_ref, dst_ref=out_ref,
                send_sem=bc_send, recv_sem=bc_recv,
                device_id=(recv_from,), device_id_type=pl.DeviceIdType.MESH)
            r.wait_recv()

        @pl.when(has_send)
        def _():
            s = pltpu.make_async_remote_copy(
                src_ref=out_ref, dst_ref=out_ref,
                send_sem=bc_send, recv_sem=bc_recv,
                device_id=(send_to,), device_id_type=pl.DeviceIdType.MESH)
            s.start()
            s.wait_send()

    return pl.pallas_call(
        body,
        out_shape=jax.ShapeDtypeStruct((B, SQ, DOUT), jnp.float32),
        in_specs=[pl.BlockSpec(memory_space=pltpu.VMEM)] * 5,
        out_specs=pl.BlockSpec(memory_space=pltpu.VMEM),
        scratch_shapes=[
            pltpu.VMEM((B, KV_KEEP, HD), jnp.float32),
            pltpu.VMEM((B, KV_KEEP, HD), jnp.float32),
            pltpu.SemaphoreType.DMA((2,)),
            pltpu.SemaphoreType.DMA((2,)),
            pltpu.SemaphoreType.DMA,
            pltpu.SemaphoreType.DMA,
        ],
    )(x, Wq, K2, V2, Wo)


# device time: 31259 ns/iter; 1.9897x vs baseline; 1.9897x over previous
import jax
import jax.numpy as jnp
from jax import lax
from jax.experimental import pallas as pl
from jax.experimental.pallas import tpu as pltpu

N_DEV = 8
B = 2
SQ = 256
DMODEL = 512
DOUT = 512
HQ = 4
DH = 64
HD = HQ * DH
SKV_SH = 256
WIN = 128
SCALE = 0.125


def kernel(x, Wq, K_ext, V_ext, Wo):
    K2 = K_ext.reshape(B, SKV_SH, HD)
    V2 = V_ext.reshape(B, SKV_SH, HD)

    def body(x_ref, wq_ref, k_ref, v_ref, wo_ref, out_ref,
             ctx_scr, ctx_rcv, l_snd, l_rcv, p1_send, p1_recv,
             bc_send, bc_recv):
        pos = lax.axis_index("i")

        @pl.when(pos <= 1)
        def _():
            off = pos * SKV_SH
            qi = lax.broadcasted_iota(jnp.int32, (SQ, SKV_SH), 0)
            kj = lax.broadcasted_iota(jnp.int32, (SQ, SKV_SH), 1) + off
            keep = jnp.abs(qi - kj) <= WIN
            for b in range(B):
                q2d = jnp.dot(x_ref[b], wq_ref[...],
                              preferred_element_type=jnp.float32)
                for h in range(HQ):
                    cs = slice(h * DH, (h + 1) * DH)
                    qh = q2d[:, cs]
                    kh = k_ref[b, :, cs]
                    vh = v_ref[b, :, cs]
                    s = lax.dot_general(
                        qh, kh, (((1,), (1,)), ((), ())),
                        preferred_element_type=jnp.float32) * SCALE
                    w = jnp.where(keep, jnp.exp(s), 0.0)
                    l_snd[b, :, h:h + 1] = jnp.sum(w, axis=1, keepdims=True)
                    ctx_scr[b, :, cs] = jnp.dot(
                        w, vh, preferred_element_type=jnp.float32)

        @pl.when(pos <= 1)
        def _():
            peer = 1 - pos
            xc = pltpu.make_async_remote_copy(
                src_ref=ctx_scr, dst_ref=ctx_rcv,
                send_sem=p1_send.at[0], recv_sem=p1_recv.at[0],
                device_id=(peer,), device_id_type=pl.DeviceIdType.MESH)
            xl = pltpu.make_async_remote_copy(
                src_ref=l_snd, dst_ref=l_rcv,
                send_sem=p1_send.at[1], recv_sem=p1_recv.at[1],
                device_id=(peer,), device_id_type=pl.DeviceIdType.MESH)
            xc.start()
            xl.start()
            xc.wait()
            xl.wait()

        has_recv = pos >= 2
        recv_from = jnp.where(
            jnp.logical_or(pos == 3, pos == 4), 0,
            jnp.where(jnp.logical_or(pos == 2, pos == 5), 1,
                      jnp.where(pos == 6, 2, 3)))
        child0 = jnp.where(pos == 0, 3,
                           jnp.where(pos == 1, 2,
                                     jnp.where(pos == 2, 6, 7)))
        child1 = jnp.where(pos == 0, 4, 5)

        for b in range(B):
            @pl.when(pos <= 1)
            def _():
                for h in range(HQ):
                    cs = slice(h * DH, (h + 1) * DH)
                    lt = l_snd[b, :, h:h + 1] + l_rcv[b, :, h:h + 1]
                    ctx_scr[b, :, cs] = (
                        ctx_scr[b, :, cs] + ctx_rcv[b, :, cs]) / lt

            @pl.when(has_recv)
            def _():
                r = pltpu.make_async_remote_copy(
                    src_ref=ctx_scr.at[b], dst_ref=ctx_scr.at[b],
                    send_sem=bc_send.at[0, b], recv_sem=bc_recv.at[b],
                    device_id=(recv_from,),
                    device_id_type=pl.DeviceIdType.MESH)
                r.wait_recv()

            @pl.when(pos <= 3)
            def _():
                s0 = pltpu.make_async_remote_copy(
                    src_ref=ctx_scr.at[b], dst_ref=ctx_scr.at[b],
                    send_sem=bc_send.at[0, b], recv_sem=bc_recv.at[b],
                    device_id=(child0,),
                    device_id_type=pl.DeviceIdType.MESH)
                s0.start()

            @pl.when(pos <= 1)
            def _():
                s1 = pltpu.make_async_remote_copy(
                    src_ref=ctx_scr.at[b], dst_ref=ctx_scr.at[b],
                    send_sem=bc_send.at[1, b], recv_sem=bc_recv.at[b],
                    device_id=(child1,),
                    device_id_type=pl.DeviceIdType.MESH)
                s1.start()

            out_ref[b] = jnp.dot(ctx_scr[b], wo_ref[...],
                                 preferred_element_type=jnp.float32)

        for b in range(B):
            @pl.when(pos <= 3)
            def _():
                d0 = pltpu.make_async_remote_copy(
                    src_ref=ctx_scr.at[b], dst_ref=ctx_scr.at[b],
                    send_sem=bc_send.at[0, b], recv_sem=bc_recv.at[b],
                    device_id=(child0,),
                    device_id_type=pl.DeviceIdType.MESH)
                d0.wait_send()

            @pl.when(pos <= 1)
            def _():
                d1 = pltpu.make_async_remote_copy(
                    src_ref=ctx_scr.at[b], dst_ref=ctx_scr.at[b],
                    send_sem=bc_send.at[1, b], recv_sem=bc_recv.at[b],
                    device_id=(child1,),
                    device_id_type=pl.DeviceIdType.MESH)
                d1.wait_send()

    return pl.pallas_call(
        body,
        out_shape=jax.ShapeDtypeStruct((B, SQ, DOUT), jnp.float32),
        in_specs=[pl.BlockSpec(memory_space=pltpu.VMEM)] * 5,
        out_specs=pl.BlockSpec(memory_space=pltpu.VMEM),
        scratch_shapes=[
            pltpu.VMEM((B, SQ, HD), jnp.float32),
            pltpu.VMEM((B, SQ, HD), jnp.float32),
            pltpu.VMEM((B, SQ, HQ), jnp.float32),
            pltpu.VMEM((B, SQ, HQ), jnp.float32),
            pltpu.SemaphoreType.DMA((2,)),
            pltpu.SemaphoreType.DMA((2,)),
            pltpu.SemaphoreType.DMA((2, B)),
            pltpu.SemaphoreType.DMA((B,)),
        ],
    )(x, Wq, K2, V2, Wo)


# device time: 27700 ns/iter; 2.2453x vs baseline; 1.1285x over previous
import jax
import jax.numpy as jnp
from jax import lax
from jax.experimental import pallas as pl
from jax.experimental.pallas import tpu as pltpu

N_DEV = 8
B = 2
SQ = 256
DMODEL = 512
DOUT = 512
HQ = 4
DH = 64
HD = HQ * DH
SKV_SH = 256
WIN = 128
REL1 = SQ + WIN - SKV_SH
SCALE = 0.125


def kernel(x, Wq, K_ext, V_ext, Wo):
    K2 = K_ext.reshape(B, SKV_SH, HD)
    V2 = V_ext.reshape(B, SKV_SH, HD)

    def body(x_ref, wq_ref, k_ref, v_ref, wo_ref, out_ref,
             ctx_scr, ctx_rcv, l_snd, l_rcv, p1_send, p1_recv,
             bc_send, bc_recv):
        pos = lax.axis_index("i")
        peer = 1 - pos

        def exchange_descr(b):
            xc = pltpu.make_async_remote_copy(
                src_ref=ctx_scr.at[b], dst_ref=ctx_rcv.at[b],
                send_sem=p1_send.at[0, b], recv_sem=p1_recv.at[0, b],
                device_id=(peer,), device_id_type=pl.DeviceIdType.MESH)
            xl = pltpu.make_async_remote_copy(
                src_ref=l_snd.at[b], dst_ref=l_rcv.at[b],
                send_sem=p1_send.at[1, b], recv_sem=p1_recv.at[1, b],
                device_id=(peer,), device_id_type=pl.DeviceIdType.MESH)
            return xc, xl

        def partial_and_exchange(kv_len, off):
            qi = lax.broadcasted_iota(jnp.int32, (SQ, kv_len), 0)
            kj = lax.broadcasted_iota(jnp.int32, (SQ, kv_len), 1) + off
            keep = jnp.abs(qi - kj) <= WIN
            for b in range(B):
                q2d = jnp.dot(x_ref[b], wq_ref[...],
                              preferred_element_type=jnp.float32)
                for h in range(HQ):
                    cs = slice(h * DH, (h + 1) * DH)
                    qh = q2d[:, cs]
                    kh = k_ref[b, 0:kv_len, cs]
                    vh = v_ref[b, 0:kv_len, cs]
                    s = lax.dot_general(
                        qh, kh, (((1,), (1,)), ((), ())),
                        preferred_element_type=jnp.float32) * SCALE
                    w = jnp.where(keep, jnp.exp(s), 0.0)
                    l_snd[b, :, h:h + 1] = jnp.sum(w, axis=1, keepdims=True)
                    ctx_scr[b, :, cs] = jnp.dot(
                        w, vh, preferred_element_type=jnp.float32)
                xc, xl = exchange_descr(b)
                xc.start()
                xl.start()

        @pl.when(pos == 0)
        def _():
            partial_and_exchange(SKV_SH, 0)

        @pl.when(pos == 1)
        def _():
            partial_and_exchange(REL1, SKV_SH)

        has_recv = pos >= 2
        recv_from = jnp.where(
            jnp.logical_or(pos == 3, pos == 4), 0,
            jnp.where(jnp.logical_or(pos == 2, pos == 5), 1,
                      jnp.where(pos == 6, 2, 3)))
        child0 = jnp.where(pos == 0, 3,
                           jnp.where(pos == 1, 2,
                                     jnp.where(pos == 2, 6, 7)))
        child1 = jnp.where(pos == 0, 4, 5)

        for b in range(B):
            @pl.when(pos <= 1)
            def _():
                xc, xl = exchange_descr(b)
                xc.wait()
                xl.wait()
                for h in range(HQ):
                    cs = slice(h * DH, (h + 1) * DH)
                    lt = l_snd[b, :, h:h + 1] + l_rcv[b, :, h:h + 1]
                    ctx_scr[b, :, cs] = (
                        ctx_scr[b, :, cs] + ctx_rcv[b, :, cs]) / lt

            @pl.when(has_recv)
            def _():
                r = pltpu.make_async_remote_copy(
                    src_ref=ctx_scr.at[b], dst_ref=ctx_scr.at[b],
                    send_sem=bc_send.at[0, b], recv_sem=bc_recv.at[b],
                    device_id=(recv_from,),
                    device_id_type=pl.DeviceIdType.MESH)
                r.wait_recv()

            @pl.when(pos <= 3)
            def _():
                s0 = pltpu.make_async_remote_copy(
                    src_ref=ctx_scr.at[b], dst_ref=ctx_scr.at[b],
                    send_sem=bc_send.at[0, b], recv_sem=bc_recv.at[b],
                    device_id=(child0,),
                    device_id_type=pl.DeviceIdType.MESH)
                s0.start()

            @pl.when(pos <= 1)
            def _():
                s1 = pltpu.make_async_remote_copy(
                    src_ref=ctx_scr.at[b], dst_ref=ctx_scr.at[b],
                    send_sem=bc_send.at[1, b], recv_sem=bc_recv.at[b],
                    device_id=(child1,),
                    device_id_type=pl.DeviceIdType.MESH)
                s1.start()

            out_ref[b] = jnp.dot(ctx_scr[b], wo_ref[...],
                                 preferred_element_type=jnp.float32)

        for b in range(B):
            @pl.when(pos <= 3)
            def _():
                d0 = pltpu.make_async_remote_copy(
                    src_ref=ctx_scr.at[b], dst_ref=ctx_scr.at[b],
                    send_sem=bc_send.at[0, b], recv_sem=bc_recv.at[b],
                    device_id=(child0,),
                    device_id_type=pl.DeviceIdType.MESH)
                d0.wait_send()

            @pl.when(pos <= 1)
            def _():
                d1 = pltpu.make_async_remote_copy(
                    src_ref=ctx_scr.at[b], dst_ref=ctx_scr.at[b],
                    send_sem=bc_send.at[1, b], recv_sem=bc_recv.at[b],
                    device_id=(child1,),
                    device_id_type=pl.DeviceIdType.MESH)
                d1.wait_send()

    return pl.pallas_call(
        body,
        out_shape=jax.ShapeDtypeStruct((B, SQ, DOUT), jnp.float32),
        in_specs=[pl.BlockSpec(memory_space=pltpu.VMEM)] * 5,
        out_specs=pl.BlockSpec(memory_space=pltpu.VMEM),
        scratch_shapes=[
            pltpu.VMEM((B, SQ, HD), jnp.float32),
            pltpu.VMEM((B, SQ, HD), jnp.float32),
            pltpu.VMEM((B, SQ, HQ), jnp.float32),
            pltpu.VMEM((B, SQ, HQ), jnp.float32),
            pltpu.SemaphoreType.DMA((2, B)),
            pltpu.SemaphoreType.DMA((2, B)),
            pltpu.SemaphoreType.DMA((2, B)),
            pltpu.SemaphoreType.DMA((B,)),
        ],
    )(x, Wq, K2, V2, Wo)


# device time: 22865 ns/iter; 2.7201x vs baseline; 1.2115x over previous
import jax
import jax.numpy as jnp
from jax import lax
from jax.experimental import pallas as pl
from jax.experimental.pallas import tpu as pltpu

N_DEV = 8
B = 2
SQ = 256
DMODEL = 512
DOUT = 512
HQ = 4
DH = 64
HD = HQ * DH
SKV_SH = 256
WIN = 128
REL1 = SQ + WIN - SKV_SH
SCALE = 0.125
RH = SQ // 2
CHUNKS = [(b, r) for b in range(B) for r in (0, RH)]
NC = len(CHUNKS)


def kernel(x, Wq, K_ext, V_ext, Wo):
    K2 = K_ext.reshape(B, SKV_SH, HD)
    V2 = V_ext.reshape(B, SKV_SH, HD)

    def body(x_ref, wq_ref, k_ref, v_ref, wo_ref, out_ref,
             ctx_scr, ctx_rcv, l_snd, l_rcv, p1_send, p1_recv,
             bc_send, bc_recv):
        pos = lax.axis_index("i")
        peer = 1 - pos

        barrier = pltpu.get_barrier_semaphore()
        partner_sets = {0: (1, 3, 4), 1: (0, 2, 5), 2: (1, 6),
                        3: (0, 7), 4: (0,), 5: (1,), 6: (2,), 7: (3,)}
        for p, partners in partner_sets.items():
            @pl.when(pos == p)
            def _(partners=partners):
                for t in partners:
                    pl.semaphore_signal(
                        barrier, inc=1, device_id=(t,),
                        device_id_type=pl.DeviceIdType.MESH)
        n_partners = jnp.where(pos <= 1, 3, jnp.where(pos <= 3, 2, 1))
        pl.semaphore_wait(barrier, n_partners)

        def exchange_descr(b):
            xl = pltpu.make_async_remote_copy(
                src_ref=l_snd.at[b], dst_ref=l_rcv.at[b],
                send_sem=p1_send.at[1, b], recv_sem=p1_recv.at[1, b],
                device_id=(peer,), device_id_type=pl.DeviceIdType.MESH)
            xc = pltpu.make_async_remote_copy(
                src_ref=ctx_scr.at[b], dst_ref=ctx_rcv.at[b],
                send_sem=p1_send.at[0, b], recv_sem=p1_recv.at[0, b],
                device_id=(peer,), device_id_type=pl.DeviceIdType.MESH)
            return xl, xc

        def partial_and_exchange(kv_len, off):
            qi = lax.broadcasted_iota(jnp.int32, (SQ, kv_len), 0)
            kj = lax.broadcasted_iota(jnp.int32, (SQ, kv_len), 1) + off
            keep = jnp.abs(qi - kj) <= WIN
            for b in range(B):
                q2d = jnp.dot(x_ref[b], wq_ref[...],
                              preferred_element_type=jnp.float32)
                for h in range(HQ):
                    cs = slice(h * DH, (h + 1) * DH)
                    qh = q2d[:, cs]
                    kh = k_ref[b, 0:kv_len, cs]
                    vh = v_ref[b, 0:kv_len, cs]
                    s = lax.dot_general(
                        qh, kh, (((1,), (1,)), ((), ())),
                        preferred_element_type=jnp.float32) * SCALE
                    w = jnp.where(keep, jnp.exp(s), 0.0)
                    l_snd[b, :, h:h + 1] = jnp.sum(w, axis=1, keepdims=True)
                    ctx_scr[b, :, cs] = jnp.dot(
                        w, vh, preferred_element_type=jnp.float32)
                xl, xc = exchange_descr(b)
                xl.start()
                xc.start()

        @pl.when(pos == 0)
        def _():
            partial_and_exchange(SKV_SH, 0)

        @pl.when(pos == 1)
        def _():
            partial_and_exchange(REL1, SKV_SH)

        has_recv = pos >= 2
        recv_from = jnp.where(
            jnp.logical_or(pos == 3, pos == 4), 0,
            jnp.where(jnp.logical_or(pos == 2, pos == 5), 1,
                      jnp.where(pos == 6, 2, 3)))
        child0 = jnp.where(pos == 0, 3,
                           jnp.where(pos == 1, 2,
                                     jnp.where(pos == 2, 6, 7)))
        child1 = jnp.where(pos == 0, 4, 5)

        def bc_descr(c, sender_slot, target):
            b, r = CHUNKS[c]
            return pltpu.make_async_remote_copy(
                src_ref=ctx_scr.at[b, pl.ds(r, RH)],
                dst_ref=ctx_scr.at[b, pl.ds(r, RH)],
                send_sem=bc_send.at[sender_slot, c],
                recv_sem=bc_recv.at[c],
                device_id=(target,), device_id_type=pl.DeviceIdType.MESH)

        for c, (b, r) in enumerate(CHUNKS):
            @pl.when(pos <= 1)
            def _():
                if r == 0:
                    xl, xc = exchange_descr(b)
                    xl.wait()
                    xc.wait()
                rs = pl.ds(r, RH)
                for h in range(HQ):
                    cs = slice(h * DH, (h + 1) * DH)
                    lt = l_snd[b, rs, h:h + 1] + l_rcv[b, rs, h:h + 1]
                    ctx_scr[b, rs, cs] = (
                        ctx_scr[b, rs, cs] + ctx_rcv[b, rs, cs]) / lt

            @pl.when(has_recv)
            def _():
                bc_descr(c, 0, recv_from).wait_recv()

            @pl.when(pos <= 3)
            def _():
                bc_descr(c, 0, child0).start()

            @pl.when(pos <= 1)
            def _():
                bc_descr(c, 1, child1).start()

            @pl.when(has_recv)
            def _():
                out_ref[b, r:r + RH] = jnp.dot(
                    ctx_scr[b, r:r + RH], wo_ref[...],
                    preferred_element_type=jnp.float32)

        @pl.when(pos <= 1)
        def _():
            for b in range(B):
                out_ref[b] = jnp.dot(ctx_scr[b], wo_ref[...],
                                     preferred_element_type=jnp.float32)

        for c in range(NC):
            @pl.when(pos <= 3)
            def _():
                bc_descr(c, 0, child0).wait_send()

            @pl.when(pos <= 1)
            def _():
                bc_descr(c, 1, child1).wait_send()

    return pl.pallas_call(
        body,
        out_shape=jax.ShapeDtypeStruct((B, SQ, DOUT), jnp.float32),
        in_specs=[pl.BlockSpec(memory_space=pltpu.VMEM)] * 5,
        out_specs=pl.BlockSpec(memory_space=pltpu.VMEM),
        scratch_shapes=[
            pltpu.VMEM((B, SQ, HD), jnp.float32),
            pltpu.VMEM((B, SQ, HD), jnp.float32),
            pltpu.VMEM((B, SQ, HQ), jnp.float32),
            pltpu.VMEM((B, SQ, HQ), jnp.float32),
            pltpu.SemaphoreType.DMA((2, B)),
            pltpu.SemaphoreType.DMA((2, B)),
            pltpu.SemaphoreType.DMA((2, NC)),
            pltpu.SemaphoreType.DMA((NC,)),
        ],
        compiler_params=pltpu.CompilerParams(collective_id=0),
    )(x, Wq, K2, V2, Wo)


# device time: 21517 ns/iter; 2.8906x vs baseline; 1.0626x over previous
import jax
import jax.numpy as jnp
from jax import lax
from jax.experimental import pallas as pl
from jax.experimental.pallas import tpu as pltpu

N_DEV = 8
B = 2
SQ = 256
DMODEL = 512
DOUT = 512
HQ = 4
DH = 64
HD = HQ * DH
SKV_SH = 256
WIN = 128
REL1 = SQ + WIN - SKV_SH
SCALE = 0.125
RH = SQ // 2
CHUNKS = [(b, r) for b in range(B) for r in (0, RH)]
NC = len(CHUNKS)


def kernel(x, Wq, K_ext, V_ext, Wo):
    K2 = K_ext.reshape(B, SKV_SH, HD)
    V2 = V_ext.reshape(B, SKV_SH, HD)

    def body(x_ref, wq_ref, k_ref, v_ref, wo_ref, out_ref,
             ctx_scr, ctx_rcv, l_snd, l_rcv, p1_send, p1_recv,
             bc_send, bc_recv):
        pos = lax.axis_index("i")
        peer = 1 - pos

        barrier = pltpu.get_barrier_semaphore()
        partner_sets = {0: (1, 3, 4), 1: (0, 2, 5), 2: (1, 6),
                        3: (0, 7), 4: (0,), 5: (1,), 6: (2,), 7: (3,)}
        for p, partners in partner_sets.items():
            @pl.when(pos == p)
            def _(partners=partners):
                for t in partners:
                    pl.semaphore_signal(
                        barrier, inc=1, device_id=(t,),
                        device_id_type=pl.DeviceIdType.MESH)
        n_partners = jnp.where(pos <= 1, 3, jnp.where(pos <= 3, 2, 1))
        pl.semaphore_wait(barrier, n_partners)

        def exchange_descr(c):
            b, r = CHUNKS[c]
            xl = pltpu.make_async_remote_copy(
                src_ref=l_snd.at[b, pl.ds(r, RH)],
                dst_ref=l_rcv.at[b, pl.ds(r, RH)],
                send_sem=p1_send.at[1, c], recv_sem=p1_recv.at[1, c],
                device_id=(peer,), device_id_type=pl.DeviceIdType.MESH)
            xc = pltpu.make_async_remote_copy(
                src_ref=ctx_scr.at[b, pl.ds(r, RH)],
                dst_ref=ctx_rcv.at[b, pl.ds(r, RH)],
                send_sem=p1_send.at[0, c], recv_sem=p1_recv.at[0, c],
                device_id=(peer,), device_id_type=pl.DeviceIdType.MESH)
            return xl, xc

        def partial_chunk(b, r, kv_len, off):
            qi = lax.broadcasted_iota(jnp.int32, (RH, kv_len), 0) + r
            kj = lax.broadcasted_iota(jnp.int32, (RH, kv_len), 1) + off
            keep = jnp.abs(qi - kj) <= WIN
            q2d = jnp.dot(x_ref[b, r:r + RH], wq_ref[...],
                          preferred_element_type=jnp.float32)
            for h in range(HQ):
                cs = slice(h * DH, (h + 1) * DH)
                s = lax.dot_general(
                    q2d[:, cs], k_ref[b, 0:kv_len, cs],
                    (((1,), (1,)), ((), ())),
                    preferred_element_type=jnp.float32) * SCALE
                w = jnp.where(keep, jnp.exp(s), 0.0)
                l_snd[b, r:r + RH, h:h + 1] = jnp.sum(w, axis=1,
                                                      keepdims=True)
                ctx_scr[b, r:r + RH, cs] = jnp.dot(
                    w, v_ref[b, 0:kv_len, cs],
                    preferred_element_type=jnp.float32)

        @pl.when(pos == 1)
        def _():
            for c, (b, r) in enumerate(CHUNKS):
                if r == RH:
                    partial_chunk(b, r, REL1, SKV_SH)
                    xl, xc = exchange_descr(c)
                    xl.start()
                    xc.start()

        has_recv = pos >= 2
        is_fwd = jnp.logical_or(pos == 2, pos == 3)
        recv_from = jnp.where(
            jnp.logical_or(pos == 3, pos == 4), 0,
            jnp.where(jnp.logical_or(pos == 2, pos == 5), 1,
                      jnp.where(pos == 6, 2, 3)))
        child0 = jnp.where(pos == 0, 3,
                           jnp.where(pos == 1, 2,
                                     jnp.where(pos == 2, 6, 7)))
        child1 = jnp.where(pos == 0, 4, 5)

        def bc_descr(c, sender_slot, target, src):
            b, r = CHUNKS[c]
            return pltpu.make_async_remote_copy(
                src_ref=src.at[b, pl.ds(r, RH)],
                dst_ref=ctx_scr.at[b, pl.ds(r, RH)],
                send_sem=bc_send.at[sender_slot, c],
                recv_sem=bc_recv.at[c],
                device_id=(target,), device_id_type=pl.DeviceIdType.MESH)

        for c, (b, r) in enumerate(CHUNKS):
            rs = pl.ds(r, RH)

            @pl.when(pos == 0)
            def _():
                partial_chunk(b, r, SKV_SH, 0)
                xl, xc = exchange_descr(c)
                xl.start()
                xc.start()
                if r == 0:
                    for h in range(HQ):
                        cs = slice(h * DH, (h + 1) * DH)
                        ctx_rcv[b, rs, cs] = (
                            ctx_scr[b, rs, cs] / l_snd[b, rs, h:h + 1])
                else:
                    xl2, xc2 = exchange_descr(c)
                    xl2.wait_recv()
                    xc2.wait_recv()
                    for h in range(HQ):
                        cs = slice(h * DH, (h + 1) * DH)
                        lt = l_snd[b, rs, h:h + 1] + l_rcv[b, rs, h:h + 1]
                        ctx_rcv[b, rs, cs] = (
                            ctx_scr[b, rs, cs] + ctx_rcv[b, rs, cs]) / lt

            @pl.when(pos == 1)
            def _():
                xl, xc = exchange_descr(c)
                xl.wait_recv()
                xc.wait_recv()
                if r == 0:
                    for h in range(HQ):
                        cs = slice(h * DH, (h + 1) * DH)
                        ctx_rcv[b, rs, cs] = (
                            ctx_rcv[b, rs, cs] / l_rcv[b, rs, h:h + 1])
                else:
                    for h in range(HQ):
                        cs = slice(h * DH, (h + 1) * DH)
                        lt = l_snd[b, rs, h:h + 1] + l_rcv[b, rs, h:h + 1]
                        ctx_rcv[b, rs, cs] = (
                            ctx_scr[b, rs, cs] + ctx_rcv[b, rs, cs]) / lt

            @pl.when(has_recv)
            def _():
                bc_descr(c, 0, recv_from, ctx_scr).wait_recv()

            @pl.when(pos <= 1)
            def _():
                bc_descr(c, 0, child0, ctx_rcv).start()
                bc_descr(c, 1, child1, ctx_rcv).start()

            @pl.when(is_fwd)
            def _():
                bc_descr(c, 0, child0, ctx_scr).start()

            @pl.when(has_recv)
            def _():
                out_ref[b, r:r + RH] = jnp.dot(
                    ctx_scr[b, r:r + RH], wo_ref[...],
                    preferred_element_type=jnp.float32)

        @pl.when(pos <= 1)
        def _():
            for b in range(B):
                out_ref[b] = jnp.dot(ctx_rcv[b], wo_ref[...],
                                     preferred_element_type=jnp.float32)

        for c, (b, r) in enumerate(CHUNKS):
            @pl.when(pos == 0)
            def _():
                xl, xc = exchange_descr(c)
                xl.wait_send()
                xc.wait_send()

            @pl.when(pos == 1)
            def _():
                if r == RH:
                    xl, xc = exchange_descr(c)
                    xl.wait_send()
                    xc.wait_send()

            @pl.when(pos <= 1)
            def _():
                bc_descr(c, 0, child0, ctx_rcv).wait_send()
                bc_descr(c, 1, child1, ctx_rcv).wait_send()

            @pl.when(is_fwd)
            def _():
                bc_descr(c, 0, child0, ctx_scr).wait_send()

    return pl.pallas_call(
        body,
        out_shape=jax.ShapeDtypeStruct((B, SQ, DOUT), jnp.float32),
        in_specs=[pl.BlockSpec(memory_space=pltpu.VMEM)] * 5,
        out_specs=pl.BlockSpec(memory_space=pltpu.VMEM),
        scratch_shapes=[
            pltpu.VMEM((B, SQ, HD), jnp.float32),
            pltpu.VMEM((B, SQ, HD), jnp.float32),
            pltpu.VMEM((B, SQ, HQ), jnp.float32),
            pltpu.VMEM((B, SQ, HQ), jnp.float32),
            pltpu.SemaphoreType.DMA((2, NC)),
            pltpu.SemaphoreType.DMA((2, NC)),
            pltpu.SemaphoreType.DMA((2, NC)),
            pltpu.SemaphoreType.DMA((NC,)),
        ],
        compiler_params=pltpu.CompilerParams(collective_id=0),
    )(x, Wq, K2, V2, Wo)


# device time: 18050 ns/iter; 3.4458x vs baseline; 1.1921x over previous
import jax
import jax.numpy as jnp
from jax import lax
from jax.experimental import pallas as pl
from jax.experimental.pallas import tpu as pltpu

N_DEV = 8
B = 2
SQ = 256
DMODEL = 512
DOUT = 512
HQ = 4
DH = 64
HD = HQ * DH
SKV_SH = 256
WIN = 128
REL1 = SQ + WIN - SKV_SH
SCALE = 0.125
RH = SQ // 2
CHUNKS = [(b, r) for b in range(B) for r in (0, RH)]
NC = len(CHUNKS)


def kernel(x, Wq, K_ext, V_ext, Wo):
    K2 = K_ext.reshape(B, SKV_SH, HD)
    V2 = V_ext.reshape(B, SKV_SH, HD)

    def body(x_ref, wq_ref, k_ref, v_ref, wo_ref, out_ref,
             ctx_scr, ctx_rcv, l_snd, l_rcv, p1_send, p1_recv,
             bc_send, bc_recv):
        pos = lax.axis_index("i")
        peer = 1 - pos

        barrier = pltpu.get_barrier_semaphore()
        partner_sets = {0: (1, 3, 4), 1: (0, 2, 5), 2: (1, 6),
                        3: (0, 7), 4: (0,), 5: (1,), 6: (2,), 7: (3,)}
        for p, partners in partner_sets.items():
            @pl.when(pos == p)
            def _(partners=partners):
                for t in partners:
                    pl.semaphore_signal(
                        barrier, inc=1, device_id=(t,),
                        device_id_type=pl.DeviceIdType.MESH)
        n_partners = jnp.where(pos <= 1, 3, jnp.where(pos <= 3, 2, 1))
        pl.semaphore_wait(barrier, n_partners)

        def exchange_descr(c):
            b, r = CHUNKS[c]
            xl = pltpu.make_async_remote_copy(
                src_ref=l_snd.at[b, pl.ds(r, RH)],
                dst_ref=l_rcv.at[b, pl.ds(r, RH)],
                send_sem=p1_send.at[1, c], recv_sem=p1_recv.at[1, c],
                device_id=(peer,), device_id_type=pl.DeviceIdType.MESH)
            xc = pltpu.make_async_remote_copy(
                src_ref=ctx_scr.at[b, pl.ds(r, RH)],
                dst_ref=ctx_rcv.at[b, pl.ds(r, RH)],
                send_sem=p1_send.at[0, c], recv_sem=p1_recv.at[0, c],
                device_id=(peer,), device_id_type=pl.DeviceIdType.MESH)
            return xl, xc

        def partial_chunk(b, r, kv_len, off):
            qi = lax.broadcasted_iota(jnp.int32, (RH, kv_len), 0) + r
            kj = lax.broadcasted_iota(jnp.int32, (RH, kv_len), 1) + off
            keep = jnp.abs(qi - kj) <= WIN
            q2d = jnp.dot(x_ref[b, r:r + RH], wq_ref[...],
                          preferred_element_type=jnp.float32)
            for h in range(HQ):
                cs = slice(h * DH, (h + 1) * DH)
                s = lax.dot_general(
                    q2d[:, cs], k_ref[b, 0:kv_len, cs],
                    (((1,), (1,)), ((), ())),
                    preferred_element_type=jnp.float32) * SCALE
                w = jnp.where(keep, jnp.exp(s), 0.0)
                l_snd[b, r:r + RH, h:h + 1] = jnp.sum(w, axis=1,
                                                      keepdims=True)
                ctx_scr[b, r:r + RH, cs] = jnp.dot(
                    w, v_ref[b, 0:kv_len, cs],
                    preferred_element_type=jnp.float32).astype(jnp.bfloat16)

        @pl.when(pos == 1)
        def _():
            for c, (b, r) in enumerate(CHUNKS):
                if r == RH:
                    partial_chunk(b, r, REL1, SKV_SH)
                    xl, xc = exchange_descr(c)
                    xl.start()
                    xc.start()

        has_recv = pos >= 2
        is_fwd = jnp.logical_or(pos == 2, pos == 3)
        recv_from = jnp.where(
            jnp.logical_or(pos == 3, pos == 4), 0,
            jnp.where(jnp.logical_or(pos == 2, pos == 5), 1,
                      jnp.where(pos == 6, 2, 3)))
        child0 = jnp.where(pos == 0, 3,
                           jnp.where(pos == 1, 2,
                                     jnp.where(pos == 2, 6, 7)))
        child1 = jnp.where(pos == 0, 4, 5)

        def bc_descr(c, sender_slot, target, src):
            b, r = CHUNKS[c]
            return pltpu.make_async_remote_copy(
                src_ref=src.at[b, pl.ds(r, RH)],
                dst_ref=ctx_scr.at[b, pl.ds(r, RH)],
                send_sem=bc_send.at[sender_slot, c],
                recv_sem=bc_recv.at[c],
                device_id=(target,), device_id_type=pl.DeviceIdType.MESH)

        for c, (b, r) in enumerate(CHUNKS):
            rs = pl.ds(r, RH)

            @pl.when(pos == 0)
            def _():
                partial_chunk(b, r, SKV_SH, 0)
                xl, xc = exchange_descr(c)
                xl.start()
                xc.start()
                if r == 0:
                    for h in range(HQ):
                        cs = slice(h * DH, (h + 1) * DH)
                        ctx_rcv[b, rs, cs] = (
                            ctx_scr[b, rs, cs].astype(jnp.float32)
                            / l_snd[b, rs, h:h + 1]).astype(jnp.bfloat16)
                else:
                    xl2, xc2 = exchange_descr(c)
                    xl2.wait_recv()
                    xc2.wait_recv()
                    for h in range(HQ):
                        cs = slice(h * DH, (h + 1) * DH)
                        lt = l_snd[b, rs, h:h + 1] + l_rcv[b, rs, h:h + 1]
                        ctx_rcv[b, rs, cs] = (
                            (ctx_scr[b, rs, cs].astype(jnp.float32)
                             + ctx_rcv[b, rs, cs].astype(jnp.float32))
                            / lt).astype(jnp.bfloat16)

            @pl.when(pos == 1)
            def _():
                xl, xc = exchange_descr(c)
                xl.wait_recv()
                xc.wait_recv()
                if r == 0:
                    for h in range(HQ):
                        cs = slice(h * DH, (h + 1) * DH)
                        ctx_rcv[b, rs, cs] = (
                            ctx_rcv[b, rs, cs].astype(jnp.float32)
                            / l_rcv[b, rs, h:h + 1]).astype(jnp.bfloat16)
                else:
                    for h in range(HQ):
                        cs = slice(h * DH, (h + 1) * DH)
                        lt = l_snd[b, rs, h:h + 1] + l_rcv[b, rs, h:h + 1]
                        ctx_rcv[b, rs, cs] = (
                            (ctx_scr[b, rs, cs].astype(jnp.float32)
                             + ctx_rcv[b, rs, cs].astype(jnp.float32))
                            / lt).astype(jnp.bfloat16)

            @pl.when(has_recv)
            def _():
                bc_descr(c, 0, recv_from, ctx_scr).wait_recv()

            @pl.when(pos <= 1)
            def _():
                bc_descr(c, 0, child0, ctx_rcv).start()
                bc_descr(c, 1, child1, ctx_rcv).start()

            @pl.when(is_fwd)
            def _():
                bc_descr(c, 0, child0, ctx_scr).start()

            @pl.when(has_recv)
            def _():
                out_ref[b, r:r + RH] = jnp.dot(
                    ctx_scr[b, r:r + RH].astype(jnp.float32), wo_ref[...],
                    preferred_element_type=jnp.float32)

        @pl.when(pos <= 1)
        def _():
            for b in range(B):
                out_ref[b] = jnp.dot(ctx_rcv[b].astype(jnp.float32),
                                     wo_ref[...],
                                     preferred_element_type=jnp.float32)

        for c, (b, r) in enumerate(CHUNKS):
            @pl.when(pos == 0)
            def _():
                xl, xc = exchange_descr(c)
                xl.wait_send()
                xc.wait_send()

            @pl.when(pos == 1)
            def _():
                if r == RH:
                    xl, xc = exchange_descr(c)
                    xl.wait_send()
                    xc.wait_send()

            @pl.when(pos <= 1)
            def _():
                bc_descr(c, 0, child0, ctx_rcv).wait_send()
                bc_descr(c, 1, child1, ctx_rcv).wait_send()

            @pl.when(is_fwd)
            def _():
                bc_descr(c, 0, child0, ctx_scr).wait_send()

    return pl.pallas_call(
        body,
        out_shape=jax.ShapeDtypeStruct((B, SQ, DOUT), jnp.float32),
        in_specs=[pl.BlockSpec(memory_space=pltpu.VMEM)] * 5,
        out_specs=pl.BlockSpec(memory_space=pltpu.VMEM),
        scratch_shapes=[
            pltpu.VMEM((B, SQ, HD), jnp.bfloat16),
            pltpu.VMEM((B, SQ, HD), jnp.bfloat16),
            pltpu.VMEM((B, SQ, HQ), jnp.float32),
            pltpu.VMEM((B, SQ, HQ), jnp.float32),
            pltpu.SemaphoreType.DMA((2, NC)),
            pltpu.SemaphoreType.DMA((2, NC)),
            pltpu.SemaphoreType.DMA((2, NC)),
            pltpu.SemaphoreType.DMA((NC,)),
        ],
        compiler_params=pltpu.CompilerParams(collective_id=0),
    )(x, Wq, K2, V2, Wo)


# device time: 14275 ns/iter; 4.3570x vs baseline; 1.2644x over previous
import jax
import jax.numpy as jnp
from jax import lax
from jax.experimental import pallas as pl
from jax.experimental.pallas import tpu as pltpu

N_DEV = 8
B = 2
SQ = 256
DMODEL = 512
DOUT = 512
HQ = 4
DH = 64
HD = HQ * DH
SKV_SH = 256
WIN = 128
REL1 = SQ + WIN - SKV_SH
SCALE = 0.125
RH = SQ // 2
CHUNKS = [(b, r) for b in range(B) for r in (0, RH)]
NC = len(CHUNKS)


def kernel(x, Wq, K_ext, V_ext, Wo):
    x = x.astype(jnp.bfloat16)
    Wq = Wq.astype(jnp.bfloat16)
    K2 = K_ext.reshape(B, SKV_SH, HD).astype(jnp.bfloat16)
    V2 = V_ext.reshape(B, SKV_SH, HD).astype(jnp.bfloat16)

    def body(x_ref, wq_ref, k_ref, v_ref, wo_ref, out_ref,
             ctx_scr, ctx_rcv, l_snd, l_rcv, p1_send, p1_recv,
             bc_send, bc_recv):
        pos = lax.axis_index("i")
        peer = 1 - pos

        barrier = pltpu.get_barrier_semaphore()
        partner_sets = {0: (1, 3, 4), 1: (0, 2, 5), 2: (1, 6),
                        3: (0, 7), 4: (0,), 5: (1,), 6: (2,), 7: (3,)}
        for p, partners in partner_sets.items():
            @pl.when(pos == p)
            def _(partners=partners):
                for t in partners:
                    pl.semaphore_signal(
                        barrier, inc=1, device_id=(t,),
                        device_id_type=pl.DeviceIdType.MESH)
        n_partners = jnp.where(pos <= 1, 3, jnp.where(pos <= 3, 2, 1))
        pl.semaphore_wait(barrier, n_partners)

        def exchange_descr(c):
            b, r = CHUNKS[c]
            xl = pltpu.make_async_remote_copy(
                src_ref=l_snd.at[b, pl.ds(r, RH)],
                dst_ref=l_rcv.at[b, pl.ds(r, RH)],
                send_sem=p1_send.at[1, c], recv_sem=p1_recv.at[1, c],
                device_id=(peer,), device_id_type=pl.DeviceIdType.MESH)
            xc = pltpu.make_async_remote_copy(
                src_ref=ctx_scr.at[b, pl.ds(r, RH)],
                dst_ref=ctx_rcv.at[b, pl.ds(r, RH)],
                send_sem=p1_send.at[0, c], recv_sem=p1_recv.at[0, c],
                device_id=(peer,), device_id_type=pl.DeviceIdType.MESH)
            return xl, xc

        def partial_chunk(b, r, kv_len, off):
            qi = lax.broadcasted_iota(jnp.int32, (RH, kv_len), 0) + r
            kj = lax.broadcasted_iota(jnp.int32, (RH, kv_len), 1) + off
            keep = jnp.abs(qi - kj) <= WIN
            q2d = jnp.dot(x_ref[b, r:r + RH], wq_ref[...],
                          preferred_element_type=jnp.float32)
            q_bf = q2d.astype(jnp.bfloat16)
            for h in range(HQ):
                cs = slice(h * DH, (h + 1) * DH)
                s = lax.dot_general(
                    q_bf[:, cs], k_ref[b, 0:kv_len, cs],
                    (((1,), (1,)), ((), ())),
                    preferred_element_type=jnp.float32) * SCALE
                w = jnp.where(keep, jnp.exp(s), 0.0)
                l_snd[b, r:r + RH, h:h + 1] = jnp.sum(w, axis=1,
                                                      keepdims=True)
                ctx_scr[b, r:r + RH, cs] = jnp.dot(
                    w.astype(jnp.bfloat16), v_ref[b, 0:kv_len, cs],
                    preferred_element_type=jnp.float32).astype(jnp.bfloat16)

        @pl.when(pos == 1)
        def _():
            for c, (b, r) in enumerate(CHUNKS):
                if r == RH:
                    partial_chunk(b, r, REL1, SKV_SH)
                    xl, xc = exchange_descr(c)
                    xl.start()
                    xc.start()

        @pl.when(pos == 0)
        def _():
            for c, (b, r) in enumerate(CHUNKS):
                partial_chunk(b, r, SKV_SH, 0)
                xl, xc = exchange_descr(c)
                xl.start()
                xc.start()
                if r == 0:
                    rs = pl.ds(r, RH)
                    for h in range(HQ):
                        cs = slice(h * DH, (h + 1) * DH)
                        ctx_rcv[b, rs, cs] = (
                            ctx_scr[b, rs, cs].astype(jnp.float32)
                            / l_snd[b, rs, h:h + 1]).astype(jnp.bfloat16)

        has_recv = pos >= 2
        is_fwd = jnp.logical_or(pos == 2, pos == 3)
        recv_from = jnp.where(
            jnp.logical_or(pos == 3, pos == 4), 0,
            jnp.where(jnp.logical_or(pos == 2, pos == 5), 1,
                      jnp.where(pos == 6, 2, 3)))
        child0 = jnp.where(pos == 0, 3,
                           jnp.where(pos == 1, 2,
                                     jnp.where(pos == 2, 6, 7)))
        child1 = jnp.where(pos == 0, 4, 5)

        def bc_descr(c, sender_slot, target, src):
            b, r = CHUNKS[c]
            return pltpu.make_async_remote_copy(
                src_ref=src.at[b, pl.ds(r, RH)],
                dst_ref=ctx_scr.at[b, pl.ds(r, RH)],
                send_sem=bc_send.at[sender_slot, c],
                recv_sem=bc_recv.at[c],
                device_id=(target,), device_id_type=pl.DeviceIdType.MESH)

        for c, (b, r) in enumerate(CHUNKS):
            rs = pl.ds(r, RH)

            if r == RH:
                @pl.when(pos == 0)
                def _():
                    xl2, xc2 = exchange_descr(c)
                    xl2.wait_recv()
                    xc2.wait_recv()
                    for h in range(HQ):
                        cs = slice(h * DH, (h + 1) * DH)
                        lt = l_snd[b, rs, h:h + 1] + l_rcv[b, rs, h:h + 1]
                        ctx_rcv[b, rs, cs] = (
                            (ctx_scr[b, rs, cs].astype(jnp.float32)
                             + ctx_rcv[b, rs, cs].astype(jnp.float32))
                            / lt).astype(jnp.bfloat16)

            @pl.when(pos == 1)
            def _():
                xl, xc = exchange_descr(c)
                xl.wait_recv()
                xc.wait_recv()
                if r == 0:
                    for h in range(HQ):
                        cs = slice(h * DH, (h + 1) * DH)
                        ctx_rcv[b, rs, cs] = (
                            ctx_rcv[b, rs, cs].astype(jnp.float32)
                            / l_rcv[b, rs, h:h + 1]).astype(jnp.bfloat16)
                else:
                    for h in range(HQ):
                        cs = slice(h * DH, (h + 1) * DH)
                        lt = l_snd[b, rs, h:h + 1] + l_rcv[b, rs, h:h + 1]
                        ctx_rcv[b, rs, cs] = (
                            (ctx_scr[b, rs, cs].astype(jnp.float32)
                             + ctx_rcv[b, rs, cs].astype(jnp.float32))
                            / lt).astype(jnp.bfloat16)

            @pl.when(has_recv)
            def _():
                bc_descr(c, 0, recv_from, ctx_scr).wait_recv()

            @pl.when(pos <= 1)
            def _():
                bc_descr(c, 0, child0, ctx_rcv).start()
                bc_descr(c, 1, child1, ctx_rcv).start()

            @pl.when(is_fwd)
            def _():
                bc_descr(c, 0, child0, ctx_scr).start()

            @pl.when(has_recv)
            def _():
                out_ref[b, r:r + RH] = jnp.dot(
                    ctx_scr[b, r:r + RH].astype(jnp.float32), wo_ref[...],
                    preferred_element_type=jnp.float32)

        @pl.when(pos <= 1)
        def _():
            for b in range(B):
                out_ref[b] = jnp.dot(ctx_rcv[b].astype(jnp.float32),
                                     wo_ref[...],
                                     preferred_element_type=jnp.float32)

        for c, (b, r) in enumerate(CHUNKS):
            @pl.when(pos == 0)
            def _():
                xl, xc = exchange_descr(c)
                xl.wait_send()
                xc.wait_send()

            @pl.when(pos == 1)
            def _():
                if r == RH:
                    xl, xc = exchange_descr(c)
                    xl.wait_send()
                    xc.wait_send()

            @pl.when(pos <= 1)
            def _():
                bc_descr(c, 0, child0, ctx_rcv).wait_send()
                bc_descr(c, 1, child1, ctx_rcv).wait_send()

            @pl.when(is_fwd)
            def _():
                bc_descr(c, 0, child0, ctx_scr).wait_send()

    return pl.pallas_call(
        body,
        out_shape=jax.ShapeDtypeStruct((B, SQ, DOUT), jnp.float32),
        in_specs=[pl.BlockSpec(memory_space=pltpu.VMEM)] * 5,
        out_specs=pl.BlockSpec(memory_space=pltpu.VMEM),
        scratch_shapes=[
            pltpu.VMEM((B, SQ, HD), jnp.bfloat16),
            pltpu.VMEM((B, SQ, HD), jnp.bfloat16),
            pltpu.VMEM((B, SQ, HQ), jnp.float32),
            pltpu.VMEM((B, SQ, HQ), jnp.float32),
            pltpu.SemaphoreType.DMA((2, NC)),
            pltpu.SemaphoreType.DMA((2, NC)),
            pltpu.SemaphoreType.DMA((2, NC)),
            pltpu.SemaphoreType.DMA((NC,)),
        ],
        compiler_params=pltpu.CompilerParams(collective_id=0),
    )(x, Wq, K2, V2, Wo)
